# baseline (speedup 1.0000x reference)
import jax
import jax.numpy as jnp
from jax import lax
from jax.experimental import pallas as pl
from jax.experimental.pallas import tpu as pltpu
from jax.experimental.pallas import tpu_sc as plsc

B = 16384
D = 32
M = 4
NC = 2
NS = 16
NW = NC * NS
PW = B // NW
NIT = 100000
C = 64
NCH = PW // C
GROUPS = C // 16

_mesh = plsc.VectorSubcoreMesh(core_axis_name="c", subcore_axis_name="s")

TCB = 128
KSLABS = 4
ITR = 25088
TGRID = ITR // TCB
_LASTB = (NIT + TCB - 1) // TCB - 1


def _tc_body(x0, x1, x2, x3, o_ref):
    o_ref[...] = jnp.concatenate(
        [x0[...].T, x1[...].T, x2[...].T, x3[...].T], axis=1)


def _in_spec(k):
    return pl.BlockSpec(
        (D, TCB), lambda b, k=k: (0, jnp.minimum(TGRID * k + b, _LASTB)))


_tc_call = pl.pallas_call(
    _tc_body,
    out_shape=jax.ShapeDtypeStruct((ITR, KSLABS * D), jnp.float32),
    grid=(TGRID,),
    in_specs=[_in_spec(k) for k in range(KSLABS)],
    out_specs=pl.BlockSpec((TCB, KSLABS * D), lambda b: (b, 0)),
)


def _body(uid_hbm, iid_hbm, taste_hbm, attn_hbm, item_hbm, ub_hbm, ib_hbm,
          out_hbm, uid_v, iid_v, iid_r, ub_v, ib_v, out_v,
          taste0, taste1, attn0, attn1, item0, item1,
          bsem, sem0, sem1):
    wid = lax.axis_index("s") * NC + lax.axis_index("c")
    base = wid * PW
    pltpu.sync_copy(uid_hbm.at[pl.ds(base, PW)], uid_v)
    pltpu.sync_copy(iid_hbm.at[pl.ds(base, PW)], iid_v)

    def mod_ids(s, carry):
        iid_r[pl.ds(s * 16, 16)] = jnp.remainder(
            iid_v[pl.ds(s * 16, 16)], jnp.full((16,), ITR, jnp.int32))
        return carry
    lax.fori_loop(0, PW // 16, mod_ids, None)

    taste_s = (taste0, taste1)
    attn_s = (attn0, attn1)
    item_s = (item0, item1)
    sems = (sem0, sem1)

    def fire(c, slot):
        cbase = c * C
        uidx = uid_v.at[pl.ds(cbase, C)]
        iidx = iid_r.at[pl.ds(cbase, C)]
        pltpu.async_copy(taste_hbm.at[uidx], taste_s[slot], sems[slot])
        pltpu.async_copy(attn_hbm.at[uidx], attn_s[slot], sems[slot])
        pltpu.async_copy(item_hbm.at[iidx], item_s[slot], sems[slot])

    def drain(slot):
        pltpu.make_async_copy(taste_hbm.at[pl.ds(0, C)], taste_s[slot],
                              sems[slot]).wait()
        pltpu.make_async_copy(attn_hbm.at[pl.ds(0, C)], attn_s[slot],
                              sems[slot]).wait()
        pltpu.make_async_copy(item_hbm.at[pl.ds(0, C)], item_s[slot],
                              sems[slot]).wait()

    cp_ub = pltpu.async_copy(ub_hbm.at[uid_v], ub_v, bsem)
    cp_ib = pltpu.async_copy(ib_hbm.at[iid_v], ib_v, bsem)
    fire(0, 0)
    fire(1, 1)
    cp_ub.wait()
    cp_ib.wait()

    def chunk_pair(it, carry):
        for slot in range(2):
            cbase = (it * 2 + slot) * C
            drain(slot)

            def group(g, carry2, cbase=cbase, slot=slot):
                taste_v, attn_v, item_b = (
                    taste_s[slot], attn_s[slot], item_s[slot])
                lane = lax.iota(jnp.int32, 16)
                rows = jnp.full((16,), g * 16, jnp.int32) + lane
                cb = (iid_v[pl.ds(cbase + g * 16, 16)]
                      // jnp.full((16,), ITR, jnp.int32)) * D
                zero = jnp.zeros((16,), jnp.float32)
                logits = [zero, zero, zero, zero]
                tdots = [zero, zero, zero, zero]
                for j in range(D):
                    sk = (jnp.full((16,), j, jnp.int32) + lane) & (D - 1)
                    iv = plsc.load_gather(item_b, [rows, cb + sk])
                    for m in range(M):
                        fc = sk + (m * D)
                        av = plsc.load_gather(attn_v, [rows, fc])
                        tv = plsc.load_gather(taste_v, [rows, fc])
                        logits[m] = logits[m] + av * iv
                        tdots[m] = tdots[m] + tv * iv
                mx = jnp.maximum(jnp.maximum(logits[0], logits[1]),
                                 jnp.maximum(logits[2], logits[3]))
                e = [jnp.exp(l - mx) for l in logits]
                num = (e[0] * tdots[0] + e[1] * tdots[1] + e[2] * tdots[2]
                       + e[3] * tdots[3])
                den = e[0] + e[1] + e[2] + e[3]
                obase = cbase + g * 16
                res = (num / den + ub_v[pl.ds(obase, 16)]
                       + ib_v[pl.ds(obase, 16)])
                out_v[pl.ds(obase, 16)] = res
                return carry2

            lax.fori_loop(0, GROUPS, group, None)

            @pl.when(it < NCH // 2 - 1)
            def _(slot=slot):
                fire(it * 2 + slot + 2, slot)
        return carry

    lax.fori_loop(0, NCH // 2, chunk_pair, None)

    pltpu.sync_copy(out_v, out_hbm.at[pl.ds(base, PW)])


_sc_call = pl.kernel(
    _body,
    out_type=jax.ShapeDtypeStruct((B,), jnp.float32),
    mesh=_mesh,
    scratch_types=[
        pltpu.VMEM((PW,), jnp.int32),
        pltpu.VMEM((PW,), jnp.int32),
        pltpu.VMEM((PW,), jnp.int32),
        pltpu.VMEM((PW,), jnp.float32),
        pltpu.VMEM((PW,), jnp.float32),
        pltpu.VMEM((PW,), jnp.float32),
        pltpu.VMEM((C, M * D), jnp.float32),
        pltpu.VMEM((C, M * D), jnp.float32),
        pltpu.VMEM((C, M * D), jnp.float32),
        pltpu.VMEM((C, M * D), jnp.float32),
        pltpu.VMEM((C, M * D), jnp.float32),
        pltpu.VMEM((C, M * D), jnp.float32),
        pltpu.SemaphoreType.DMA,
        pltpu.SemaphoreType.DMA,
        pltpu.SemaphoreType.DMA,
    ],
    compiler_params=pltpu.CompilerParams(needs_layout_passes=False,
                                         use_tc_tiling_on_sc=False),
)


def kernel(user_ids, item_ids, taste_table, attention_table, item_table,
           user_biases, item_biases):
    uid = user_ids.astype(jnp.int32)
    iid = item_ids.astype(jnp.int32)
    ub = user_biases.reshape(-1)
    ib = item_biases.reshape(-1)
    it = item_table.T
    item_rows = _tc_call(it, it, it, it)
    return _sc_call(uid, iid, taste_table, attention_table, item_rows, ub, ib)


# speedup vs baseline: 1.0171x; 1.0171x over previous
import jax
import jax.numpy as jnp
from jax import lax
from jax.experimental import pallas as pl
from jax.experimental.pallas import tpu as pltpu
from jax.experimental.pallas import tpu_sc as plsc

B = 16384
D = 32
M = 4
NC = 2
NS = 16
NW = NC * NS
PW = B // NW
NIT = 100000
C = 64
NCH = PW // C
GROUPS = C // 16

_mesh = plsc.VectorSubcoreMesh(core_axis_name="c", subcore_axis_name="s")

TCB = 128
KSLABS = 4
ITR = 25088
TGRID = ITR // TCB
_LASTB = (NIT + TCB - 1) // TCB - 1


def _tc_body(x0, x1, x2, x3, o_ref):
    xs = jnp.concatenate([x0[...], x1[...], x2[...], x3[...]], axis=0)
    eye = jnp.asarray(
        lax.broadcasted_iota(jnp.int32, (128, 128), 0)
        == lax.broadcasted_iota(jnp.int32, (128, 128), 1), jnp.float32)
    o_ref[...] = lax.dot_general(
        xs, eye, (((0,), (0,)), ((), ())),
        preferred_element_type=jnp.float32)


def _in_spec(k):
    return pl.BlockSpec(
        (D, TCB), lambda b, k=k: (0, jnp.minimum(TGRID * k + b, _LASTB)))


_tc_call = pl.pallas_call(
    _tc_body,
    out_shape=jax.ShapeDtypeStruct((ITR, KSLABS * D), jnp.float32),
    grid=(TGRID,),
    in_specs=[_in_spec(k) for k in range(KSLABS)],
    out_specs=pl.BlockSpec((TCB, KSLABS * D), lambda b: (b, 0)),
)


def _body(uid_hbm, iid_hbm, taste_hbm, attn_hbm, item_hbm, ub_hbm, ib_hbm,
          out_hbm, uid_v, iid_v, iid_r, ub_v, ib_v, out_v,
          taste0, taste1, attn0, attn1, item0, item1,
          bsem, sem0, sem1):
    wid = lax.axis_index("s") * NC + lax.axis_index("c")
    base = wid * PW
    pltpu.sync_copy(uid_hbm.at[pl.ds(base, PW)], uid_v)
    pltpu.sync_copy(iid_hbm.at[pl.ds(base, PW)], iid_v)

    def mod_ids(s, carry):
        iid_r[pl.ds(s * 16, 16)] = jnp.remainder(
            iid_v[pl.ds(s * 16, 16)], jnp.full((16,), ITR, jnp.int32))
        return carry
    lax.fori_loop(0, PW // 16, mod_ids, None)

    taste_s = (taste0, taste1)
    attn_s = (attn0, attn1)
    item_s = (item0, item1)
    sems = (sem0, sem1)

    def fire(c, slot):
        cbase = c * C
        uidx = uid_v.at[pl.ds(cbase, C)]
        iidx = iid_r.at[pl.ds(cbase, C)]
        pltpu.async_copy(taste_hbm.at[uidx], taste_s[slot], sems[slot])
        pltpu.async_copy(attn_hbm.at[uidx], attn_s[slot], sems[slot])
        pltpu.async_copy(item_hbm.at[iidx], item_s[slot], sems[slot])

    def drain(slot):
        pltpu.make_async_copy(taste_hbm.at[pl.ds(0, C)], taste_s[slot],
                              sems[slot]).wait()
        pltpu.make_async_copy(attn_hbm.at[pl.ds(0, C)], attn_s[slot],
                              sems[slot]).wait()
        pltpu.make_async_copy(item_hbm.at[pl.ds(0, C)], item_s[slot],
                              sems[slot]).wait()

    cp_ub = pltpu.async_copy(ub_hbm.at[uid_v], ub_v, bsem)
    cp_ib = pltpu.async_copy(ib_hbm.at[iid_v], ib_v, bsem)
    fire(0, 0)
    fire(1, 1)
    cp_ub.wait()
    cp_ib.wait()

    def chunk_pair(it, carry):
        for slot in range(2):
            cbase = (it * 2 + slot) * C
            drain(slot)

            def group(g, carry2, cbase=cbase, slot=slot):
                taste_v, attn_v, item_b = (
                    taste_s[slot], attn_s[slot], item_s[slot])
                lane = lax.iota(jnp.int32, 16)
                rows = jnp.full((16,), g * 16, jnp.int32) + lane
                cb = (iid_v[pl.ds(cbase + g * 16, 16)]
                      // jnp.full((16,), ITR, jnp.int32)) * D
                zero = jnp.zeros((16,), jnp.float32)
                logits = [zero, zero, zero, zero]
                tdots = [zero, zero, zero, zero]
                for j in range(D):
                    sk = (jnp.full((16,), j, jnp.int32) + lane) & (D - 1)
                    iv = plsc.load_gather(item_b, [rows, cb + sk])
                    for m in range(M):
                        fc = sk + (m * D)
                        av = plsc.load_gather(attn_v, [rows, fc])
                        tv = plsc.load_gather(taste_v, [rows, fc])
                        logits[m] = logits[m] + av * iv
                        tdots[m] = tdots[m] + tv * iv
                mx = jnp.maximum(jnp.maximum(logits[0], logits[1]),
                                 jnp.maximum(logits[2], logits[3]))
                e = [jnp.exp(l - mx) for l in logits]
                num = (e[0] * tdots[0] + e[1] * tdots[1] + e[2] * tdots[2]
                       + e[3] * tdots[3])
                den = e[0] + e[1] + e[2] + e[3]
                obase = cbase + g * 16
                res = (num / den + ub_v[pl.ds(obase, 16)]
                       + ib_v[pl.ds(obase, 16)])
                out_v[pl.ds(obase, 16)] = res
                return carry2

            lax.fori_loop(0, GROUPS, group, None)

            @pl.when(it < NCH // 2 - 1)
            def _(slot=slot):
                fire(it * 2 + slot + 2, slot)
        return carry

    lax.fori_loop(0, NCH // 2, chunk_pair, None)

    pltpu.sync_copy(out_v, out_hbm.at[pl.ds(base, PW)])


_sc_call = pl.kernel(
    _body,
    out_type=jax.ShapeDtypeStruct((B,), jnp.float32),
    mesh=_mesh,
    scratch_types=[
        pltpu.VMEM((PW,), jnp.int32),
        pltpu.VMEM((PW,), jnp.int32),
        pltpu.VMEM((PW,), jnp.int32),
        pltpu.VMEM((PW,), jnp.float32),
        pltpu.VMEM((PW,), jnp.float32),
        pltpu.VMEM((PW,), jnp.float32),
        pltpu.VMEM((C, M * D), jnp.float32),
        pltpu.VMEM((C, M * D), jnp.float32),
        pltpu.VMEM((C, M * D), jnp.float32),
        pltpu.VMEM((C, M * D), jnp.float32),
        pltpu.VMEM((C, M * D), jnp.float32),
        pltpu.VMEM((C, M * D), jnp.float32),
        pltpu.SemaphoreType.DMA,
        pltpu.SemaphoreType.DMA,
        pltpu.SemaphoreType.DMA,
    ],
    compiler_params=pltpu.CompilerParams(needs_layout_passes=False,
                                         use_tc_tiling_on_sc=False),
)


def kernel(user_ids, item_ids, taste_table, attention_table, item_table,
           user_biases, item_biases):
    uid = user_ids.astype(jnp.int32)
    iid = item_ids.astype(jnp.int32)
    ub = user_biases.reshape(-1)
    ib = item_biases.reshape(-1)
    it = item_table.T
    item_rows = _tc_call(it, it, it, it)
    return _sc_call(uid, iid, taste_table, attention_table, item_rows, ub, ib)


# speedup vs baseline: 2.9219x; 2.8729x over previous
import jax
import jax.numpy as jnp
from jax import lax
from jax.experimental import pallas as pl
from jax.experimental.pallas import tpu as pltpu
from jax.experimental.pallas import tpu_sc as plsc

B = 16384
D = 32
M = 4
NC = 2
NS = 16
NW = NC * NS
PW = B // NW
NIT = 100000
C = 64
NCH = PW // C
GROUPS = C // 16

_mesh = plsc.VectorSubcoreMesh(core_axis_name="c", subcore_axis_name="s")

TCB = 3584
KSLABS = 4
ITR = 25088
TGRID = ITR // TCB
_LASTB = (NIT + TCB - 1) // TCB - 1


def _tc_body(x0, x1, x2, x3, o_ref):
    xs = jnp.concatenate([x0[...], x1[...], x2[...], x3[...]], axis=0)
    eye = jnp.asarray(
        lax.broadcasted_iota(jnp.int32, (128, 128), 0)
        == lax.broadcasted_iota(jnp.int32, (128, 128), 1), jnp.float32)
    o_ref[...] = lax.dot_general(
        xs, eye, (((0,), (0,)), ((), ())),
        preferred_element_type=jnp.float32)


def _in_spec(k):
    return pl.BlockSpec(
        (D, TCB), lambda b, k=k: (0, jnp.minimum(TGRID * k + b, _LASTB)))


_tc_call = pl.pallas_call(
    _tc_body,
    out_shape=jax.ShapeDtypeStruct((ITR, KSLABS * D), jnp.float32),
    grid=(TGRID,),
    in_specs=[_in_spec(k) for k in range(KSLABS)],
    out_specs=pl.BlockSpec((TCB, KSLABS * D), lambda b: (b, 0)),
)


def _body(uid_hbm, iid_hbm, taste_hbm, attn_hbm, item_hbm, ub_hbm, ib_hbm,
          out_hbm, uid_v, iid_v, iid_r, ub_v, ib_v, out_v,
          taste0, taste1, attn0, attn1, item0, item1,
          bsem, sem0, sem1):
    wid = lax.axis_index("s") * NC + lax.axis_index("c")
    base = wid * PW
    pltpu.sync_copy(uid_hbm.at[pl.ds(base, PW)], uid_v)
    pltpu.sync_copy(iid_hbm.at[pl.ds(base, PW)], iid_v)

    def mod_ids(s, carry):
        iid_r[pl.ds(s * 16, 16)] = jnp.remainder(
            iid_v[pl.ds(s * 16, 16)], jnp.full((16,), ITR, jnp.int32))
        return carry
    lax.fori_loop(0, PW // 16, mod_ids, None)

    taste_s = (taste0, taste1)
    attn_s = (attn0, attn1)
    item_s = (item0, item1)
    sems = (sem0, sem1)

    def fire(c, slot):
        cbase = c * C
        uidx = uid_v.at[pl.ds(cbase, C)]
        iidx = iid_r.at[pl.ds(cbase, C)]
        pltpu.async_copy(taste_hbm.at[uidx], taste_s[slot], sems[slot])
        pltpu.async_copy(attn_hbm.at[uidx], attn_s[slot], sems[slot])
        pltpu.async_copy(item_hbm.at[iidx], item_s[slot], sems[slot])

    def drain(slot):
        pltpu.make_async_copy(taste_hbm.at[pl.ds(0, C)], taste_s[slot],
                              sems[slot]).wait()
        pltpu.make_async_copy(attn_hbm.at[pl.ds(0, C)], attn_s[slot],
                              sems[slot]).wait()
        pltpu.make_async_copy(item_hbm.at[pl.ds(0, C)], item_s[slot],
                              sems[slot]).wait()

    cp_ub = pltpu.async_copy(ub_hbm.at[uid_v], ub_v, bsem)
    cp_ib = pltpu.async_copy(ib_hbm.at[iid_v], ib_v, bsem)
    fire(0, 0)
    fire(1, 1)
    cp_ub.wait()
    cp_ib.wait()

    def chunk_pair(it, carry):
        for slot in range(2):
            cbase = (it * 2 + slot) * C
            drain(slot)

            def group(g, carry2, cbase=cbase, slot=slot):
                taste_v, attn_v, item_b = (
                    taste_s[slot], attn_s[slot], item_s[slot])
                lane = lax.iota(jnp.int32, 16)
                rows = jnp.full((16,), g * 16, jnp.int32) + lane
                cb = (iid_v[pl.ds(cbase + g * 16, 16)]
                      // jnp.full((16,), ITR, jnp.int32)) * D
                zero = jnp.zeros((16,), jnp.float32)
                logits = [zero, zero, zero, zero]
                tdots = [zero, zero, zero, zero]
                for j in range(D):
                    sk = (jnp.full((16,), j, jnp.int32) + lane) & (D - 1)
                    iv = plsc.load_gather(item_b, [rows, cb + sk])
                    for m in range(M):
                        fc = sk + (m * D)
                        av = plsc.load_gather(attn_v, [rows, fc])
                        tv = plsc.load_gather(taste_v, [rows, fc])
                        logits[m] = logits[m] + av * iv
                        tdots[m] = tdots[m] + tv * iv
                mx = jnp.maximum(jnp.maximum(logits[0], logits[1]),
                                 jnp.maximum(logits[2], logits[3]))
                e = [jnp.exp(l - mx) for l in logits]
                num = (e[0] * tdots[0] + e[1] * tdots[1] + e[2] * tdots[2]
                       + e[3] * tdots[3])
                den = e[0] + e[1] + e[2] + e[3]
                obase = cbase + g * 16
                res = (num / den + ub_v[pl.ds(obase, 16)]
                       + ib_v[pl.ds(obase, 16)])
                out_v[pl.ds(obase, 16)] = res
                return carry2

            lax.fori_loop(0, GROUPS, group, None)

            @pl.when(it < NCH // 2 - 1)
            def _(slot=slot):
                fire(it * 2 + slot + 2, slot)
        return carry

    lax.fori_loop(0, NCH // 2, chunk_pair, None)

    pltpu.sync_copy(out_v, out_hbm.at[pl.ds(base, PW)])


_sc_call = pl.kernel(
    _body,
    out_type=jax.ShapeDtypeStruct((B,), jnp.float32),
    mesh=_mesh,
    scratch_types=[
        pltpu.VMEM((PW,), jnp.int32),
        pltpu.VMEM((PW,), jnp.int32),
        pltpu.VMEM((PW,), jnp.int32),
        pltpu.VMEM((PW,), jnp.float32),
        pltpu.VMEM((PW,), jnp.float32),
        pltpu.VMEM((PW,), jnp.float32),
        pltpu.VMEM((C, M * D), jnp.float32),
        pltpu.VMEM((C, M * D), jnp.float32),
        pltpu.VMEM((C, M * D), jnp.float32),
        pltpu.VMEM((C, M * D), jnp.float32),
        pltpu.VMEM((C, M * D), jnp.float32),
        pltpu.VMEM((C, M * D), jnp.float32),
        pltpu.SemaphoreType.DMA,
        pltpu.SemaphoreType.DMA,
        pltpu.SemaphoreType.DMA,
    ],
    compiler_params=pltpu.CompilerParams(needs_layout_passes=False,
                                         use_tc_tiling_on_sc=False),
)


def kernel(user_ids, item_ids, taste_table, attention_table, item_table,
           user_biases, item_biases):
    uid = user_ids.astype(jnp.int32)
    iid = item_ids.astype(jnp.int32)
    ub = user_biases.reshape(-1)
    ib = item_biases.reshape(-1)
    it = item_table.T
    item_rows = _tc_call(it, it, it, it)
    return _sc_call(uid, iid, taste_table, attention_table, item_rows, ub, ib)
